# baseline (speedup 1.0000x reference)
import functools

import jax
import jax.numpy as jnp
from jax import lax
from jax.experimental import pallas as pl
from jax.experimental.pallas import tpu as pltpu
from jax.experimental.pallas import tpu_sc as plsc

N = 10000
E = 320000
D = 128
ED = 16
MAX_R = 4
NPAD = 10240
NW = 32
EPW = E // NW
EPP = 10240
K = 64
NCHUNK = EPP // K

_f32 = jnp.float32
_i32 = jnp.int32

_mesh = plsc.VectorSubcoreMesh(core_axis_name="c", subcore_axis_name="s")
_sc_params = pltpu.CompilerParams(needs_layout_passes=False)


def _edge_mlp_body(ef, w1, b1, w2, b2, out):
    h = jnp.dot(ef[...], w1[...], preferred_element_type=_f32) + b1[...]
    h = jax.nn.gelu(h)
    s = jnp.dot(h, w2[...], preferred_element_type=_f32) + b2[...]
    out[...] = jax.nn.sigmoid(s)


def _edge_mlp(edge_feats, W1, b1, W2, b2):
    BE = 4000
    return pl.pallas_call(
        _edge_mlp_body,
        grid=(E // BE,),
        in_specs=[
            pl.BlockSpec((BE, ED), lambda i: (i, 0)),
            pl.BlockSpec((ED, ED), lambda i: (0, 0)),
            pl.BlockSpec((1, ED), lambda i: (0, 0)),
            pl.BlockSpec((ED, 1), lambda i: (0, 0)),
            pl.BlockSpec((1, 1), lambda i: (0, 0)),
        ],
        out_specs=pl.BlockSpec((BE, 1), lambda i: (i, 0)),
        out_shape=jax.ShapeDtypeStruct((E, 1), _f32),
    )(edge_feats, W1, b1.reshape(1, ED), W2, b2.reshape(1, 1))


@functools.partial(
    pl.kernel,
    out_type=jax.ShapeDtypeStruct((NW, NPAD), _f32),
    mesh=_mesh,
    compiler_params=_sc_params,
    scratch_types=[
        pltpu.VMEM((EPW,), _i32),
        pltpu.VMEM((EPW,), _i32),
        pltpu.VMEM((EPW,), _f32),
        pltpu.VMEM((NPAD,), _f32),
    ],
)
def _deg_kernel(row_r, col_r, ew_r, out, rowv, colv, ewv, degl):
    cid = lax.axis_index("c")
    sid = lax.axis_index("s")
    wid = sid * 2 + cid

    def _zero(t, _):
        degl[pl.ds(t * 16, 16)] = jnp.zeros((16,), _f32)
        return 0
    lax.fori_loop(0, NPAD // 16, _zero, 0)

    pltpu.sync_copy(row_r.at[wid], rowv)
    pltpu.sync_copy(col_r.at[wid], colv)
    pltpu.sync_copy(ew_r.at[wid], ewv)

    def _edges(t, _):
        r = rowv[pl.ds(t * 16, 16)]
        c = colv[pl.ds(t * 16, 16)]
        w = ewv[pl.ds(t * 16, 16)]
        wm = jnp.where(r != c, w, jnp.zeros((16,), _f32))
        plsc.addupdate_scatter(degl, [c], wm)
        return 0
    lax.fori_loop(0, EPW // 16, _edges, 0)
    pltpu.sync_copy(degl, out.at[wid])


def _dis_body(dp, dis_o, dis2_o):
    deg = jnp.sum(dp[...], axis=0) + 1.0
    dis_o[...] = lax.rsqrt(deg)
    dis2_o[...] = 1.0 / deg


def _dis(degp):
    spec = pl.BlockSpec((NPAD // 128, 128), lambda: (0, 0))
    return pl.pallas_call(
        _dis_body,
        in_specs=[pl.BlockSpec((NW, NPAD // 128, 128), lambda: (0, 0, 0))],
        out_specs=[spec, spec],
        out_shape=[jax.ShapeDtypeStruct((NPAD // 128, 128), _f32),
                   jax.ShapeDtypeStruct((NPAD // 128, 128), _f32)],
    )(degp.reshape(NW, NPAD // 128, 128))


@functools.partial(
    pl.kernel,
    out_type=jax.ShapeDtypeStruct((NW, EPW), _f32),
    mesh=_mesh,
    compiler_params=_sc_params,
    scratch_types=[
        pltpu.VMEM((EPW,), _i32),
        pltpu.VMEM((EPW,), _i32),
        pltpu.VMEM((EPW,), _f32),
        pltpu.VMEM((EPW,), _f32),
        pltpu.VMEM((NPAD,), _f32),
    ],
)
def _norm_kernel(row_r, col_r, ew_r, dis_r, out, rowv, colv, ewv, normv, disv):
    cid = lax.axis_index("c")
    sid = lax.axis_index("s")
    wid = sid * 2 + cid
    pltpu.sync_copy(row_r.at[wid], rowv)
    pltpu.sync_copy(col_r.at[wid], colv)
    pltpu.sync_copy(ew_r.at[wid], ewv)
    pltpu.sync_copy(dis_r, disv)

    def _edges(t, _):
        r = rowv[pl.ds(t * 16, 16)]
        c = colv[pl.ds(t * 16, 16)]
        w = ewv[pl.ds(t * 16, 16)]
        a = plsc.load_gather(disv, [r])
        b = plsc.load_gather(disv, [c])
        wm = jnp.where(r != c, w, jnp.zeros((16,), _f32))
        normv[pl.ds(t * 16, 16)] = a * wm * b
        return 0
    lax.fori_loop(0, EPW // 16, _edges, 0)
    pltpu.sync_copy(normv, out.at[wid])


@functools.partial(
    pl.kernel,
    out_type=[jax.ShapeDtypeStruct((NPAD, D), _f32),
              jax.ShapeDtypeStruct((NPAD, D), _f32)],
    mesh=_mesh,
    compiler_params=_sc_params,
    scratch_types=[
        pltpu.VMEM((4, 3, K), _i32),
        pltpu.VMEM((K, D), _f32),
        pltpu.VMEM((K, D), _f32),
        pltpu.SemaphoreType.DMA,
        pltpu.SemaphoreType.DMA,
        pltpu.SemaphoreType.DMA,
        pltpu.SemaphoreType.DMA,
        pltpu.SemaphoreType.DMA,
        pltpu.SemaphoreType.DMA,
        pltpu.SemaphoreType.DMA,
        pltpu.SemaphoreType.DMA,
        pltpu.VMEM_SHARED((NPAD, D), _f32),
    ],
)
def _hop_kernel(h_r, sh_r, ed_r, p0, p1, edb, rows0, rows1, esem0, esem1,
                esem2, esem3, gsem0, gsem1, ssem0, ssem1, acc):
    cid = lax.axis_index("c")
    sid = lax.axis_index("s")
    wid = sid * 2 + cid
    npt = NPAD // 16
    rows = (rows0, rows1)
    esem = (esem0, esem1, esem2, esem3)
    gsem = (gsem0, gsem1)
    ssem = (ssem0, ssem1)

    pltpu.sync_copy(sh_r.at[pl.ds(sid * npt, npt)],
                    acc.at[pl.ds(sid * npt, npt)])
    plsc.subcore_barrier()

    def _start_ed(c, e):
        pltpu.async_copy(ed_r.at[wid, c], edb.at[e], esem[e])

    def _wait_ed(e):
        pltpu.make_async_copy(ed_r.at[wid, 0], edb.at[e], esem[e]).wait()

    def _start_g(e, s):
        pltpu.async_copy(h_r.at[edb.at[e, 0]], rows[s], gsem[s])

    def _wait_g(s):
        pltpu.make_async_copy(h_r.at[edb.at[0, 0]], rows[s], gsem[s]).wait()

    def _start_sc(s, e):
        pltpu.async_copy(rows[s], acc.at[edb.at[e, 1]], ssem[s], add=True)

    def _wait_sc(s):
        pltpu.make_async_copy(h_r.at[edb.at[0, 0]], rows[s], ssem[s]).wait()

    def _scale(s, e):
        buf = rows[s]

        def _grp(g, _):
            i0 = g * 4
            for u in range(4):
                nb = plsc.load_gather(
                    edb.at[e, 2], [jnp.full((16,), i0 + u, _i32)])
                nv = lax.bitcast_convert_type(nb, _f32)
                for j in range(D // 16):
                    sl = pl.ds(j * 16, 16)
                    buf[i0 + u, sl] = buf[i0 + u, sl] * nv
            return 0
        lax.fori_loop(0, K // 4, _grp, 0)

    def _body(c, s, e):
        _wait_ed((e + 1) % 4)
        _wait_sc(1 - s)
        _start_g((e + 1) % 4, 1 - s)
        _wait_g(s)
        _scale(s, e)
        _start_sc(s, e)
        _start_ed(c + 2, (e + 2) % 4)

    pltpu.sync_copy(ed_r.at[wid, 0], edb.at[0])
    _start_g(0, 0)
    _start_ed(1, 1)
    _wait_ed(1)
    _start_g(1, 1)
    _wait_g(0)
    _scale(0, 0)
    _start_sc(0, 0)
    _start_ed(2, 2)
    _wait_ed(2)
    _wait_sc(0)
    _start_g(2, 0)
    _wait_g(1)
    _scale(1, 1)
    _start_sc(1, 1)
    _start_ed(3, 3)

    def _quad(g, _):
        c0 = 4 * g + 2
        _body(c0, 0, 2)
        _body(c0 + 1, 1, 3)
        _body(c0 + 2, 0, 0)
        _body(c0 + 3, 1, 1)
        return 0
    lax.fori_loop(0, (NCHUNK - 4) // 4, _quad, 0)
    _wait_ed(3)
    _wait_sc(1)
    _start_g(3, 1)
    _wait_g(0)
    _scale(0, 2)
    _start_sc(0, 2)
    _wait_g(1)
    _scale(1, 3)
    _start_sc(1, 3)
    _wait_sc(0)
    _wait_sc(1)
    plsc.subcore_barrier()

    @pl.when(cid == 0)
    def _():
        pltpu.sync_copy(acc.at[pl.ds(sid * npt, npt)],
                        p0.at[pl.ds(sid * npt, npt)])

    @pl.when(cid == 1)
    def _():
        pltpu.sync_copy(acc.at[pl.ds(sid * npt, npt)],
                        p1.at[pl.ds(sid * npt, npt)])


def _combine_body(p0, p1, d2, hout, shout):
    h = p0[...] + p1[...]
    hout[...] = h
    shout[...] = 0.5 * d2[...] * h


def _combine(p0, p1, d2b):
    BC = 1280
    spec = pl.BlockSpec((BC, D), lambda i: (i, 0))
    return pl.pallas_call(
        _combine_body,
        grid=(NPAD // BC,),
        in_specs=[spec, spec, spec],
        out_specs=[spec, spec],
        out_shape=[jax.ShapeDtypeStruct((NPAD, D), _f32),
                   jax.ShapeDtypeStruct((NPAD, D), _f32)],
    )(p0, p1, d2b)


def _channels_body(x_r, h1_r, h2_r, h4_r, wch_r, bch_r, apl_r, acl_r, apb_r,
                   acb_r, wout_r, bout_r, out_r):
    act = jax.nn.gelu
    x = x_r[...]
    h1 = h1_r[...]
    h2 = h2_r[...]
    h4 = h4_r[...]
    pre_low = jnp.sum(act(x) * apl_r[...], axis=-1, keepdims=True)
    pre_band = jnp.sum(act(x) * apb_r[...], axis=-1, keepdims=True)
    zs = [x, h1, h2, h4, x - h1, h1 - h2, h2 - h4]
    outs = []
    scores = []
    for ci in range(7):
        o = (jnp.dot(zs[ci], wch_r[ci], preferred_element_type=_f32)
             + bch_r[ci:ci + 1, :])
        if ci < 4:
            s = jnp.sum(act(o) * acl_r[...], axis=-1, keepdims=True) + pre_low
        else:
            s = jnp.sum(act(o) * acb_r[...], axis=-1, keepdims=True) + pre_band
        scores.append(act(s))
        outs.append(o)
    S = jnp.concatenate(scores, axis=-1)
    m = jnp.max(S, axis=-1, keepdims=True)
    ex = jnp.exp(S - m)
    alpha = ex / jnp.sum(ex, axis=-1, keepdims=True)
    comb = outs[0] * alpha[:, 0:1]
    for ci in range(1, 7):
        comb = comb + outs[ci] * alpha[:, ci:ci + 1]
    out_r[...] = (jnp.dot(comb, wout_r[0:D], preferred_element_type=_f32)
                  + jnp.dot(x, wout_r[D:2 * D], preferred_element_type=_f32)
                  + bout_r[...])


def _channels(x, h1, h2, h4, Wch, bch, apl, acl, apb, acb, Wout, bout):
    BN = 400
    row_spec = pl.BlockSpec((BN, D), lambda i: (i, 0))
    vec_spec = pl.BlockSpec((1, D), lambda i: (0, 0))
    return pl.pallas_call(
        _channels_body,
        grid=(N // BN,),
        in_specs=[
            row_spec, row_spec, row_spec, row_spec,
            pl.BlockSpec((7, D, D), lambda i: (0, 0, 0)),
            pl.BlockSpec((7, D), lambda i: (0, 0)),
            vec_spec, vec_spec, vec_spec, vec_spec,
            pl.BlockSpec((2 * D, D), lambda i: (0, 0)),
            vec_spec,
        ],
        out_specs=row_spec,
        out_shape=jax.ShapeDtypeStruct((N, D), _f32),
    )(x, h1, h2, h4, Wch, bch, apl, acl, apb, acb, Wout, bout.reshape(1, D))


def kernel(x, edge_index, edge_feats, W1, b1, W2, b2, Wch, bch, att_pre_low,
           att_channel_low, att_pre_band, att_channel_band, Wout, bout):
    row = edge_index[0]
    col = edge_index[1]
    row2d = row.reshape(NW, EPW)
    col2d = col.reshape(NW, EPW)

    ew = _edge_mlp(edge_feats, W1, b1, W2, b2)
    ew2d = ew.reshape(NW, EPW)

    degp = _deg_kernel(row2d, col2d, ew2d)
    dis_t, dis2_t = _dis(degp)
    dis = dis_t.reshape(NPAD)
    dis2 = dis2_t.reshape(NPAD)

    norm2d = _norm_kernel(row2d, col2d, ew2d, dis)
    pad = ((0, 0), (0, EPP - EPW))
    rowp = jnp.pad(row2d, pad)
    colp = jnp.pad(col2d, pad)
    normp = jnp.pad(norm2d, pad)
    ed = jnp.concatenate(
        [rowp.reshape(NW, NCHUNK, 1, K),
         colp.reshape(NW, NCHUNK, 1, K),
         lax.bitcast_convert_type(normp, _i32).reshape(NW, NCHUNK, 1, K)],
        axis=2)

    x_pad = jnp.pad(x, ((0, NPAD - N), (0, 0)))
    d2b = jnp.broadcast_to(dis2[:, None], (NPAD, D))
    h = x_pad
    sh = 0.5 * dis2[:, None] * x_pad
    hs = []
    for _ in range(MAX_R):
        p0, p1 = _hop_kernel(h, sh, ed)
        h, sh = _combine(p0, p1, d2b)
        hs.append(h)

    return _channels(x, hs[0], hs[1], hs[3], Wch, bch, att_pre_low,
                     att_channel_low, att_pre_band, att_channel_band,
                     Wout, bout)


# speedup vs baseline: 1.8864x; 1.8864x over previous
import functools

import jax
import jax.numpy as jnp
from jax import lax
from jax.experimental import pallas as pl
from jax.experimental.pallas import tpu as pltpu
from jax.experimental.pallas import tpu_sc as plsc

N = 10000
E = 320000
D = 128
ED = 16
MAX_R = 4
NPAD = 10240
NW = 32
EPW = E // NW
EPP = 10240
K = 64
NCHUNK = EPP // K

_f32 = jnp.float32
_i32 = jnp.int32

_mesh = plsc.VectorSubcoreMesh(core_axis_name="c", subcore_axis_name="s")
_sc_params = pltpu.CompilerParams(needs_layout_passes=False)


def _edge_mlp_body(ef, w1, b1, w2, b2, out):
    h = jnp.dot(ef[...], w1[...], preferred_element_type=_f32) + b1[...]
    h = jax.nn.gelu(h)
    s = jnp.dot(h, w2[...], preferred_element_type=_f32) + b2[...]
    out[...] = jax.nn.sigmoid(s)


def _edge_mlp(edge_feats, W1, b1, W2, b2):
    BE = 4000
    return pl.pallas_call(
        _edge_mlp_body,
        grid=(E // BE,),
        in_specs=[
            pl.BlockSpec((BE, ED), lambda i: (i, 0)),
            pl.BlockSpec((ED, ED), lambda i: (0, 0)),
            pl.BlockSpec((1, ED), lambda i: (0, 0)),
            pl.BlockSpec((ED, 1), lambda i: (0, 0)),
            pl.BlockSpec((1, 1), lambda i: (0, 0)),
        ],
        out_specs=pl.BlockSpec((BE, 1), lambda i: (i, 0)),
        out_shape=jax.ShapeDtypeStruct((E, 1), _f32),
    )(edge_feats, W1, b1.reshape(1, ED), W2, b2.reshape(1, 1))


@functools.partial(
    pl.kernel,
    out_type=jax.ShapeDtypeStruct((NW, NPAD), _f32),
    mesh=_mesh,
    compiler_params=_sc_params,
    scratch_types=[
        pltpu.VMEM((EPW,), _i32),
        pltpu.VMEM((EPW,), _i32),
        pltpu.VMEM((EPW,), _f32),
        pltpu.VMEM((NPAD,), _f32),
    ],
)
def _deg_kernel(row_r, col_r, ew_r, out, rowv, colv, ewv, degl):
    cid = lax.axis_index("c")
    sid = lax.axis_index("s")
    wid = sid * 2 + cid

    def _zero(t, _):
        degl[pl.ds(t * 16, 16)] = jnp.zeros((16,), _f32)
        return 0
    lax.fori_loop(0, NPAD // 16, _zero, 0)

    pltpu.sync_copy(row_r.at[wid], rowv)
    pltpu.sync_copy(col_r.at[wid], colv)
    pltpu.sync_copy(ew_r.at[wid], ewv)

    def _edges(t, _):
        r = rowv[pl.ds(t * 16, 16)]
        c = colv[pl.ds(t * 16, 16)]
        w = ewv[pl.ds(t * 16, 16)]
        wm = jnp.where(r != c, w, jnp.zeros((16,), _f32))
        plsc.addupdate_scatter(degl, [c], wm)
        return 0
    lax.fori_loop(0, EPW // 16, _edges, 0)
    pltpu.sync_copy(degl, out.at[wid])


def _dis_body(dp, dis_o, dis2_o):
    deg = jnp.sum(dp[...], axis=0) + 1.0
    dis_o[...] = lax.rsqrt(deg)
    dis2_o[...] = 1.0 / deg


def _dis(degp):
    spec = pl.BlockSpec((NPAD // 128, 128), lambda: (0, 0))
    return pl.pallas_call(
        _dis_body,
        in_specs=[pl.BlockSpec((NW, NPAD // 128, 128), lambda: (0, 0, 0))],
        out_specs=[spec, spec],
        out_shape=[jax.ShapeDtypeStruct((NPAD // 128, 128), _f32),
                   jax.ShapeDtypeStruct((NPAD // 128, 128), _f32)],
    )(degp.reshape(NW, NPAD // 128, 128))


@functools.partial(
    pl.kernel,
    out_type=jax.ShapeDtypeStruct((NW, EPW), _f32),
    mesh=_mesh,
    compiler_params=_sc_params,
    scratch_types=[
        pltpu.VMEM((EPW,), _i32),
        pltpu.VMEM((EPW,), _i32),
        pltpu.VMEM((EPW,), _f32),
        pltpu.VMEM((EPW,), _f32),
        pltpu.VMEM((NPAD,), _f32),
    ],
)
def _norm_kernel(row_r, col_r, ew_r, dis_r, out, rowv, colv, ewv, normv, disv):
    cid = lax.axis_index("c")
    sid = lax.axis_index("s")
    wid = sid * 2 + cid
    pltpu.sync_copy(row_r.at[wid], rowv)
    pltpu.sync_copy(col_r.at[wid], colv)
    pltpu.sync_copy(ew_r.at[wid], ewv)
    pltpu.sync_copy(dis_r, disv)

    def _edges(t, _):
        r = rowv[pl.ds(t * 16, 16)]
        c = colv[pl.ds(t * 16, 16)]
        w = ewv[pl.ds(t * 16, 16)]
        a = plsc.load_gather(disv, [r])
        b = plsc.load_gather(disv, [c])
        wm = jnp.where(r != c, w, jnp.zeros((16,), _f32))
        normv[pl.ds(t * 16, 16)] = a * wm * b
        return 0
    lax.fori_loop(0, EPW // 16, _edges, 0)
    pltpu.sync_copy(normv, out.at[wid])


@functools.partial(
    pl.kernel,
    out_type=[jax.ShapeDtypeStruct((NPAD, D), _f32),
              jax.ShapeDtypeStruct((NPAD, D), _f32)],
    mesh=_mesh,
    compiler_params=_sc_params,
    scratch_types=[
        pltpu.VMEM((4, 3, K), _i32),
        pltpu.VMEM((K, D), _f32),
        pltpu.VMEM((K, D), _f32),
        pltpu.SemaphoreType.DMA,
        pltpu.SemaphoreType.DMA,
        pltpu.SemaphoreType.DMA,
        pltpu.SemaphoreType.DMA,
        pltpu.SemaphoreType.DMA,
        pltpu.SemaphoreType.DMA,
        pltpu.SemaphoreType.DMA,
        pltpu.SemaphoreType.DMA,
        pltpu.VMEM_SHARED((NPAD, D), _f32),
    ],
)
def _hop_kernel(h_r, sh_r, ed_r, p0, p1, edb, rows0, rows1, esem0, esem1,
                esem2, esem3, gsem0, gsem1, ssem0, ssem1, acc):
    cid = lax.axis_index("c")
    sid = lax.axis_index("s")
    wid = sid * 2 + cid
    npt = NPAD // 16
    rows = (rows0, rows1)
    esem = (esem0, esem1, esem2, esem3)
    gsem = (gsem0, gsem1)
    ssem = (ssem0, ssem1)

    pltpu.sync_copy(sh_r.at[pl.ds(sid * npt, npt)],
                    acc.at[pl.ds(sid * npt, npt)])
    plsc.subcore_barrier()

    def _start_ed(c, e):
        pltpu.async_copy(ed_r.at[wid, c], edb.at[e], esem[e])

    def _wait_ed(e):
        pltpu.make_async_copy(ed_r.at[wid, 0], edb.at[e], esem[e]).wait()

    def _start_g(e, s):
        pltpu.async_copy(h_r.at[edb.at[e, 0]], rows[s], gsem[s])

    def _wait_g(s):
        pltpu.make_async_copy(h_r.at[edb.at[0, 0]], rows[s], gsem[s]).wait()

    def _start_sc(s, e):
        pltpu.async_copy(rows[s], acc.at[edb.at[e, 1]], ssem[s], add=True)

    def _wait_sc(s):
        pltpu.make_async_copy(h_r.at[edb.at[0, 0]], rows[s], ssem[s]).wait()

    def _scale(s, e):
        buf = rows[s]

        def _grp(g, _):
            i0 = g * 4
            for u in range(4):
                nb = plsc.load_gather(
                    edb.at[e, 2], [jnp.full((16,), i0 + u, _i32)])
                nv = lax.bitcast_convert_type(nb, _f32)
                for j in range(D // 16):
                    sl = pl.ds(j * 16, 16)
                    buf[i0 + u, sl] = buf[i0 + u, sl] * nv
            return 0
        lax.fori_loop(0, K // 4, _grp, 0)

    def _body(c, s, e):
        _wait_ed((e + 1) % 4)
        _wait_sc(1 - s)
        _start_g((e + 1) % 4, 1 - s)
        _wait_g(s)
        _scale(s, e)
        _start_sc(s, e)
        _start_ed(c + 2, (e + 2) % 4)

    pltpu.sync_copy(ed_r.at[wid, 0], edb.at[0])
    _start_g(0, 0)
    _start_ed(1, 1)
    _wait_ed(1)
    _start_g(1, 1)
    _wait_g(0)
    _scale(0, 0)
    _start_sc(0, 0)
    _start_ed(2, 2)
    _wait_ed(2)
    _wait_sc(0)
    _start_g(2, 0)
    _wait_g(1)
    _scale(1, 1)
    _start_sc(1, 1)
    _start_ed(3, 3)

    def _quad(g, _):
        c0 = 4 * g + 2
        _body(c0, 0, 2)
        _body(c0 + 1, 1, 3)
        _body(c0 + 2, 0, 0)
        _body(c0 + 3, 1, 1)
        return 0
    lax.fori_loop(0, (NCHUNK - 4) // 4, _quad, 0)
    _wait_ed(3)
    _wait_sc(1)
    _start_g(3, 1)
    _wait_g(0)
    _scale(0, 2)
    _start_sc(0, 2)
    _wait_g(1)
    _scale(1, 3)
    _start_sc(1, 3)
    _wait_sc(0)
    _wait_sc(1)
    plsc.subcore_barrier()

    @pl.when(cid == 0)
    def _():
        pltpu.sync_copy(acc.at[pl.ds(sid * npt, npt)],
                        p0.at[pl.ds(sid * npt, npt)])

    @pl.when(cid == 1)
    def _():
        pltpu.sync_copy(acc.at[pl.ds(sid * npt, npt)],
                        p1.at[pl.ds(sid * npt, npt)])


def _combine_body(p0, p1, d2, hout, shout):
    h = p0[...] + p1[...]
    hout[...] = h
    shout[...] = 0.5 * d2[...] * h


def _combine(p0, p1, d2b):
    BC = 1280
    spec = pl.BlockSpec((BC, D), lambda i: (i, 0))
    return pl.pallas_call(
        _combine_body,
        grid=(NPAD // BC,),
        in_specs=[spec, spec, spec],
        out_specs=[spec, spec],
        out_shape=[jax.ShapeDtypeStruct((NPAD, D), _f32),
                   jax.ShapeDtypeStruct((NPAD, D), _f32)],
    )(p0, p1, d2b)


def _channels_body(x_r, h1_r, h2_r, h4_r, wch_r, bch_r, apl_r, acl_r, apb_r,
                   acb_r, wout_r, bout_r, out_r):
    act = jax.nn.gelu
    x = x_r[...]
    h1 = h1_r[...]
    h2 = h2_r[...]
    h4 = h4_r[...]
    pre_low = jnp.sum(act(x) * apl_r[...], axis=-1, keepdims=True)
    pre_band = jnp.sum(act(x) * apb_r[...], axis=-1, keepdims=True)
    zs = [x, h1, h2, h4, x - h1, h1 - h2, h2 - h4]
    outs = []
    scores = []
    for ci in range(7):
        o = (jnp.dot(zs[ci], wch_r[ci], preferred_element_type=_f32)
             + bch_r[ci:ci + 1, :])
        if ci < 4:
            s = jnp.sum(act(o) * acl_r[...], axis=-1, keepdims=True) + pre_low
        else:
            s = jnp.sum(act(o) * acb_r[...], axis=-1, keepdims=True) + pre_band
        scores.append(act(s))
        outs.append(o)
    S = jnp.concatenate(scores, axis=-1)
    m = jnp.max(S, axis=-1, keepdims=True)
    ex = jnp.exp(S - m)
    alpha = ex / jnp.sum(ex, axis=-1, keepdims=True)
    comb = outs[0] * alpha[:, 0:1]
    for ci in range(1, 7):
        comb = comb + outs[ci] * alpha[:, ci:ci + 1]
    out_r[...] = (jnp.dot(comb, wout_r[0:D], preferred_element_type=_f32)
                  + jnp.dot(x, wout_r[D:2 * D], preferred_element_type=_f32)
                  + bout_r[...])


def _channels(x, h1, h2, h4, Wch, bch, apl, acl, apb, acb, Wout, bout):
    BN = 400
    row_spec = pl.BlockSpec((BN, D), lambda i: (i, 0))
    vec_spec = pl.BlockSpec((1, D), lambda i: (0, 0))
    return pl.pallas_call(
        _channels_body,
        grid=(N // BN,),
        in_specs=[
            row_spec, row_spec, row_spec, row_spec,
            pl.BlockSpec((7, D, D), lambda i: (0, 0, 0)),
            pl.BlockSpec((7, D), lambda i: (0, 0)),
            vec_spec, vec_spec, vec_spec, vec_spec,
            pl.BlockSpec((2 * D, D), lambda i: (0, 0)),
            vec_spec,
        ],
        out_specs=row_spec,
        out_shape=jax.ShapeDtypeStruct((N, D), _f32),
    )(x, h1, h2, h4, Wch, bch, apl, acl, apb, acb, Wout, bout.reshape(1, D))


def kernel(x, edge_index, edge_feats, W1, b1, W2, b2, Wch, bch, att_pre_low,
           att_channel_low, att_pre_band, att_channel_band, Wout, bout):
    row = edge_index[0]
    col = edge_index[1]
    row2d = row.reshape(NW, EPW)
    col2d = col.reshape(NW, EPW)

    ew = _edge_mlp(edge_feats, W1, b1, W2, b2)
    ew2d = ew.reshape(NW, EPW)

    degp = _deg_kernel(row2d, col2d, ew2d)
    dis_t, dis2_t = _dis(degp)
    dis = dis_t.reshape(NPAD)
    dis2 = dis2_t.reshape(NPAD)

    norm2d = _norm_kernel(row2d, col2d, ew2d, dis)
    npad_e = EPP - EPW
    pad = ((0, 0), (0, npad_e))
    pad_ids = jnp.broadcast_to(N + jnp.arange(npad_e, dtype=_i32)[None, :],
                               (NW, npad_e))
    rowp = jnp.concatenate([row2d, pad_ids], axis=1)
    colp = jnp.concatenate([col2d, pad_ids], axis=1)
    normp = jnp.pad(norm2d, pad)
    ed = jnp.concatenate(
        [rowp.reshape(NW, NCHUNK, 1, K),
         colp.reshape(NW, NCHUNK, 1, K),
         lax.bitcast_convert_type(normp, _i32).reshape(NW, NCHUNK, 1, K)],
        axis=2)

    x_pad = jnp.pad(x, ((0, NPAD - N), (0, 0)))
    d2b = jnp.broadcast_to(dis2[:, None], (NPAD, D))
    h = x_pad
    sh = 0.5 * dis2[:, None] * x_pad
    hs = []
    for _ in range(MAX_R):
        p0, p1 = _hop_kernel(h, sh, ed)
        h, sh = _combine(p0, p1, d2b)
        hs.append(h)

    return _channels(x, hs[0], hs[1], hs[3], Wch, bch, att_pre_low,
                     att_channel_low, att_pre_band, att_channel_band,
                     Wout, bout)


# speedup vs baseline: 2.1280x; 1.1281x over previous
import functools

import jax
import jax.numpy as jnp
from jax import lax
from jax.experimental import pallas as pl
from jax.experimental.pallas import tpu as pltpu
from jax.experimental.pallas import tpu_sc as plsc

N = 10000
E = 320000
D = 128
ED = 16
MAX_R = 4
NPAD = 10240
NW = 32
EPW = E // NW
EPP = 10240
K = 128
NCHUNK = EPP // K

_f32 = jnp.float32
_i32 = jnp.int32

_mesh = plsc.VectorSubcoreMesh(core_axis_name="c", subcore_axis_name="s")
_sc_params = pltpu.CompilerParams(needs_layout_passes=False)


def _edge_mlp_body(ef, w1, b1, w2, b2, out):
    h = jnp.dot(ef[...], w1[...], preferred_element_type=_f32) + b1[...]
    h = jax.nn.gelu(h)
    s = jnp.dot(h, w2[...], preferred_element_type=_f32) + b2[...]
    out[...] = jax.nn.sigmoid(s)


def _edge_mlp(edge_feats, W1, b1, W2, b2):
    BE = 4000
    return pl.pallas_call(
        _edge_mlp_body,
        grid=(E // BE,),
        in_specs=[
            pl.BlockSpec((BE, ED), lambda i: (i, 0)),
            pl.BlockSpec((ED, ED), lambda i: (0, 0)),
            pl.BlockSpec((1, ED), lambda i: (0, 0)),
            pl.BlockSpec((ED, 1), lambda i: (0, 0)),
            pl.BlockSpec((1, 1), lambda i: (0, 0)),
        ],
        out_specs=pl.BlockSpec((BE, 1), lambda i: (i, 0)),
        out_shape=jax.ShapeDtypeStruct((E, 1), _f32),
    )(edge_feats, W1, b1.reshape(1, ED), W2, b2.reshape(1, 1))


@functools.partial(
    pl.kernel,
    out_type=jax.ShapeDtypeStruct((NW, NPAD), _f32),
    mesh=_mesh,
    compiler_params=_sc_params,
    scratch_types=[
        pltpu.VMEM((EPW,), _i32),
        pltpu.VMEM((EPW,), _i32),
        pltpu.VMEM((EPW,), _f32),
        pltpu.VMEM((NPAD,), _f32),
    ],
)
def _deg_kernel(row_r, col_r, ew_r, out, rowv, colv, ewv, degl):
    cid = lax.axis_index("c")
    sid = lax.axis_index("s")
    wid = sid * 2 + cid

    def _zero(t, _):
        degl[pl.ds(t * 16, 16)] = jnp.zeros((16,), _f32)
        return 0
    lax.fori_loop(0, NPAD // 16, _zero, 0)

    pltpu.sync_copy(row_r.at[wid], rowv)
    pltpu.sync_copy(col_r.at[wid], colv)
    pltpu.sync_copy(ew_r.at[wid], ewv)

    def _edges(t, _):
        r = rowv[pl.ds(t * 16, 16)]
        c = colv[pl.ds(t * 16, 16)]
        w = ewv[pl.ds(t * 16, 16)]
        wm = jnp.where(r != c, w, jnp.zeros((16,), _f32))
        plsc.addupdate_scatter(degl, [c], wm)
        return 0
    lax.fori_loop(0, EPW // 16, _edges, 0)
    pltpu.sync_copy(degl, out.at[wid])


def _dis_body(dp, dis_o, dis2_o):
    deg = jnp.sum(dp[...], axis=0) + 1.0
    dis_o[...] = lax.rsqrt(deg)
    dis2_o[...] = 1.0 / deg


def _dis(degp):
    spec = pl.BlockSpec((NPAD // 128, 128), lambda: (0, 0))
    return pl.pallas_call(
        _dis_body,
        in_specs=[pl.BlockSpec((NW, NPAD // 128, 128), lambda: (0, 0, 0))],
        out_specs=[spec, spec],
        out_shape=[jax.ShapeDtypeStruct((NPAD // 128, 128), _f32),
                   jax.ShapeDtypeStruct((NPAD // 128, 128), _f32)],
    )(degp.reshape(NW, NPAD // 128, 128))


@functools.partial(
    pl.kernel,
    out_type=jax.ShapeDtypeStruct((NW, EPW), _f32),
    mesh=_mesh,
    compiler_params=_sc_params,
    scratch_types=[
        pltpu.VMEM((EPW,), _i32),
        pltpu.VMEM((EPW,), _i32),
        pltpu.VMEM((EPW,), _f32),
        pltpu.VMEM((EPW,), _f32),
        pltpu.VMEM((NPAD,), _f32),
    ],
)
def _norm_kernel(row_r, col_r, ew_r, dis_r, out, rowv, colv, ewv, normv, disv):
    cid = lax.axis_index("c")
    sid = lax.axis_index("s")
    wid = sid * 2 + cid
    pltpu.sync_copy(row_r.at[wid], rowv)
    pltpu.sync_copy(col_r.at[wid], colv)
    pltpu.sync_copy(ew_r.at[wid], ewv)
    pltpu.sync_copy(dis_r, disv)

    def _edges(t, _):
        r = rowv[pl.ds(t * 16, 16)]
        c = colv[pl.ds(t * 16, 16)]
        w = ewv[pl.ds(t * 16, 16)]
        a = plsc.load_gather(disv, [r])
        b = plsc.load_gather(disv, [c])
        wm = jnp.where(r != c, w, jnp.zeros((16,), _f32))
        normv[pl.ds(t * 16, 16)] = a * wm * b
        return 0
    lax.fori_loop(0, EPW // 16, _edges, 0)
    pltpu.sync_copy(normv, out.at[wid])


@functools.partial(
    pl.kernel,
    out_type=[jax.ShapeDtypeStruct((NPAD, D), _f32),
              jax.ShapeDtypeStruct((NPAD, D), _f32)],
    mesh=_mesh,
    compiler_params=_sc_params,
    scratch_types=[
        pltpu.VMEM((4, 3, K), _i32),
        pltpu.VMEM((K, D), _f32),
        pltpu.VMEM((K, D), _f32),
        pltpu.SemaphoreType.DMA,
        pltpu.SemaphoreType.DMA,
        pltpu.SemaphoreType.DMA,
        pltpu.SemaphoreType.DMA,
        pltpu.SemaphoreType.DMA,
        pltpu.SemaphoreType.DMA,
        pltpu.SemaphoreType.DMA,
        pltpu.SemaphoreType.DMA,
        pltpu.VMEM_SHARED((NPAD, D), _f32),
    ],
)
def _hop_kernel(h_r, sh_r, ed_r, p0, p1, edb, rows0, rows1, esem0, esem1,
                esem2, esem3, gsem0, gsem1, ssem0, ssem1, acc):
    cid = lax.axis_index("c")
    sid = lax.axis_index("s")
    wid = sid * 2 + cid
    npt = NPAD // 16
    rows = (rows0, rows1)
    esem = (esem0, esem1, esem2, esem3)
    gsem = (gsem0, gsem1)
    ssem = (ssem0, ssem1)

    pltpu.sync_copy(sh_r.at[pl.ds(sid * npt, npt)],
                    acc.at[pl.ds(sid * npt, npt)])
    plsc.subcore_barrier()

    def _start_ed(c, e):
        pltpu.async_copy(ed_r.at[wid, c], edb.at[e], esem[e])

    def _wait_ed(e):
        pltpu.make_async_copy(ed_r.at[wid, 0], edb.at[e], esem[e]).wait()

    def _start_g(e, s):
        pltpu.async_copy(h_r.at[edb.at[e, 0]], rows[s], gsem[s])

    def _wait_g(s):
        pltpu.make_async_copy(h_r.at[edb.at[0, 0]], rows[s], gsem[s]).wait()

    def _start_sc(s, e):
        pltpu.async_copy(rows[s], acc.at[edb.at[e, 1]], ssem[s], add=True)

    def _wait_sc(s):
        pltpu.make_async_copy(h_r.at[edb.at[0, 0]], rows[s], ssem[s]).wait()

    def _scale(s, e):
        buf = rows[s]

        def _grp(g, _):
            i0 = g * 4
            for u in range(4):
                nb = plsc.load_gather(
                    edb.at[e, 2], [jnp.full((16,), i0 + u, _i32)])
                nv = lax.bitcast_convert_type(nb, _f32)
                for j in range(D // 16):
                    sl = pl.ds(j * 16, 16)
                    buf[i0 + u, sl] = buf[i0 + u, sl] * nv
            return 0
        lax.fori_loop(0, K // 4, _grp, 0)

    def _body(c, s, e):
        _wait_ed((e + 1) % 4)
        _wait_sc(1 - s)
        _start_g((e + 1) % 4, 1 - s)
        _wait_g(s)
        _scale(s, e)
        _start_sc(s, e)
        _start_ed(c + 2, (e + 2) % 4)

    pltpu.sync_copy(ed_r.at[wid, 0], edb.at[0])
    _start_g(0, 0)
    _start_ed(1, 1)
    _wait_ed(1)
    _start_g(1, 1)
    _wait_g(0)
    _scale(0, 0)
    _start_sc(0, 0)
    _start_ed(2, 2)
    _wait_ed(2)
    _wait_sc(0)
    _start_g(2, 0)
    _wait_g(1)
    _scale(1, 1)
    _start_sc(1, 1)
    _start_ed(3, 3)

    def _quad(g, _):
        c0 = 4 * g + 2
        _body(c0, 0, 2)
        _body(c0 + 1, 1, 3)
        _body(c0 + 2, 0, 0)
        _body(c0 + 3, 1, 1)
        return 0
    lax.fori_loop(0, (NCHUNK - 4) // 4, _quad, 0)
    _wait_ed(3)
    _wait_sc(1)
    _start_g(3, 1)
    _wait_g(0)
    _scale(0, 2)
    _start_sc(0, 2)
    _wait_g(1)
    _scale(1, 3)
    _start_sc(1, 3)
    _wait_sc(0)
    _wait_sc(1)
    plsc.subcore_barrier()

    @pl.when(cid == 0)
    def _():
        pltpu.sync_copy(acc.at[pl.ds(sid * npt, npt)],
                        p0.at[pl.ds(sid * npt, npt)])

    @pl.when(cid == 1)
    def _():
        pltpu.sync_copy(acc.at[pl.ds(sid * npt, npt)],
                        p1.at[pl.ds(sid * npt, npt)])


def _combine_body(p0, p1, d2, hout, shout):
    h = p0[...] + p1[...]
    hout[...] = h
    shout[...] = 0.5 * d2[...] * h


def _combine(p0, p1, d2b):
    BC = 1280
    spec = pl.BlockSpec((BC, D), lambda i: (i, 0))
    return pl.pallas_call(
        _combine_body,
        grid=(NPAD // BC,),
        in_specs=[spec, spec, spec],
        out_specs=[spec, spec],
        out_shape=[jax.ShapeDtypeStruct((NPAD, D), _f32),
                   jax.ShapeDtypeStruct((NPAD, D), _f32)],
    )(p0, p1, d2b)


def _channels_body(x_r, h1_r, h2_r, h4_r, wch_r, bch_r, apl_r, acl_r, apb_r,
                   acb_r, wout_r, bout_r, out_r):
    act = jax.nn.gelu
    x = x_r[...]
    h1 = h1_r[...]
    h2 = h2_r[...]
    h4 = h4_r[...]
    pre_low = jnp.sum(act(x) * apl_r[...], axis=-1, keepdims=True)
    pre_band = jnp.sum(act(x) * apb_r[...], axis=-1, keepdims=True)
    zs = [x, h1, h2, h4, x - h1, h1 - h2, h2 - h4]
    outs = []
    scores = []
    for ci in range(7):
        o = (jnp.dot(zs[ci], wch_r[ci], preferred_element_type=_f32)
             + bch_r[ci:ci + 1, :])
        if ci < 4:
            s = jnp.sum(act(o) * acl_r[...], axis=-1, keepdims=True) + pre_low
        else:
            s = jnp.sum(act(o) * acb_r[...], axis=-1, keepdims=True) + pre_band
        scores.append(act(s))
        outs.append(o)
    S = jnp.concatenate(scores, axis=-1)
    m = jnp.max(S, axis=-1, keepdims=True)
    ex = jnp.exp(S - m)
    alpha = ex / jnp.sum(ex, axis=-1, keepdims=True)
    comb = outs[0] * alpha[:, 0:1]
    for ci in range(1, 7):
        comb = comb + outs[ci] * alpha[:, ci:ci + 1]
    out_r[...] = (jnp.dot(comb, wout_r[0:D], preferred_element_type=_f32)
                  + jnp.dot(x, wout_r[D:2 * D], preferred_element_type=_f32)
                  + bout_r[...])


def _channels(x, h1, h2, h4, Wch, bch, apl, acl, apb, acb, Wout, bout):
    BN = 400
    row_spec = pl.BlockSpec((BN, D), lambda i: (i, 0))
    vec_spec = pl.BlockSpec((1, D), lambda i: (0, 0))
    return pl.pallas_call(
        _channels_body,
        grid=(N // BN,),
        in_specs=[
            row_spec, row_spec, row_spec, row_spec,
            pl.BlockSpec((7, D, D), lambda i: (0, 0, 0)),
            pl.BlockSpec((7, D), lambda i: (0, 0)),
            vec_spec, vec_spec, vec_spec, vec_spec,
            pl.BlockSpec((2 * D, D), lambda i: (0, 0)),
            vec_spec,
        ],
        out_specs=row_spec,
        out_shape=jax.ShapeDtypeStruct((N, D), _f32),
    )(x, h1, h2, h4, Wch, bch, apl, acl, apb, acb, Wout, bout.reshape(1, D))


def kernel(x, edge_index, edge_feats, W1, b1, W2, b2, Wch, bch, att_pre_low,
           att_channel_low, att_pre_band, att_channel_band, Wout, bout):
    row = edge_index[0]
    col = edge_index[1]
    row2d = row.reshape(NW, EPW)
    col2d = col.reshape(NW, EPW)

    ew = _edge_mlp(edge_feats, W1, b1, W2, b2)
    ew2d = ew.reshape(NW, EPW)

    degp = _deg_kernel(row2d, col2d, ew2d)
    dis_t, dis2_t = _dis(degp)
    dis = dis_t.reshape(NPAD)
    dis2 = dis2_t.reshape(NPAD)

    norm2d = _norm_kernel(row2d, col2d, ew2d, dis)
    npad_e = EPP - EPW
    pad = ((0, 0), (0, npad_e))
    pad_ids = jnp.broadcast_to(N + jnp.arange(npad_e, dtype=_i32)[None, :],
                               (NW, npad_e))
    rowp = jnp.concatenate([row2d, pad_ids], axis=1)
    colp = jnp.concatenate([col2d, pad_ids], axis=1)
    normp = jnp.pad(norm2d, pad)
    ed = jnp.concatenate(
        [rowp.reshape(NW, NCHUNK, 1, K),
         colp.reshape(NW, NCHUNK, 1, K),
         lax.bitcast_convert_type(normp, _i32).reshape(NW, NCHUNK, 1, K)],
        axis=2)

    x_pad = jnp.pad(x, ((0, NPAD - N), (0, 0)))
    d2b = jnp.broadcast_to(dis2[:, None], (NPAD, D))
    h = x_pad
    sh = 0.5 * dis2[:, None] * x_pad
    hs = []
    for _ in range(MAX_R):
        p0, p1 = _hop_kernel(h, sh, ed)
        h, sh = _combine(p0, p1, d2b)
        hs.append(h)

    return _channels(x, hs[0], hs[1], hs[3], Wch, bch, att_pre_low,
                     att_channel_low, att_pre_band, att_channel_band,
                     Wout, bout)


# speedup vs baseline: 2.1739x; 1.0216x over previous
import functools

import jax
import jax.numpy as jnp
from jax import lax
from jax.experimental import pallas as pl
from jax.experimental.pallas import tpu as pltpu
from jax.experimental.pallas import tpu_sc as plsc

N = 10000
E = 320000
D = 128
ED = 16
MAX_R = 4
NPAD = 10240
NW = 32
EPW = E // NW
EPP = 10240
K = 128
NCHUNK = EPP // K

_f32 = jnp.float32
_i32 = jnp.int32

_mesh = plsc.VectorSubcoreMesh(core_axis_name="c", subcore_axis_name="s")
_sc_params = pltpu.CompilerParams(needs_layout_passes=False)


def _edge_mlp_body(ef, w1, b1, w2, b2, out):
    h = jnp.dot(ef[...], w1[...], preferred_element_type=_f32) + b1[...]
    h = jax.nn.gelu(h)
    s = jnp.dot(h, w2[...], preferred_element_type=_f32) + b2[...]
    out[...] = jax.nn.sigmoid(s)


def _edge_mlp(edge_feats, W1, b1, W2, b2):
    BE = 4000
    return pl.pallas_call(
        _edge_mlp_body,
        grid=(E // BE,),
        in_specs=[
            pl.BlockSpec((BE, ED), lambda i: (i, 0)),
            pl.BlockSpec((ED, ED), lambda i: (0, 0)),
            pl.BlockSpec((1, ED), lambda i: (0, 0)),
            pl.BlockSpec((ED, 1), lambda i: (0, 0)),
            pl.BlockSpec((1, 1), lambda i: (0, 0)),
        ],
        out_specs=pl.BlockSpec((BE, 1), lambda i: (i, 0)),
        out_shape=jax.ShapeDtypeStruct((E, 1), _f32),
    )(edge_feats, W1, b1.reshape(1, ED), W2, b2.reshape(1, 1))


@functools.partial(
    pl.kernel,
    out_type=jax.ShapeDtypeStruct((NW, NPAD), _f32),
    mesh=_mesh,
    compiler_params=_sc_params,
    scratch_types=[
        pltpu.VMEM((EPW,), _i32),
        pltpu.VMEM((EPW,), _i32),
        pltpu.VMEM((EPW,), _f32),
        pltpu.VMEM((NPAD,), _f32),
    ],
)
def _deg_kernel(row_r, col_r, ew_r, out, rowv, colv, ewv, degl):
    cid = lax.axis_index("c")
    sid = lax.axis_index("s")
    wid = sid * 2 + cid

    def _zero(t, _):
        degl[pl.ds(t * 16, 16)] = jnp.zeros((16,), _f32)
        return 0
    lax.fori_loop(0, NPAD // 16, _zero, 0)

    pltpu.sync_copy(row_r.at[wid], rowv)
    pltpu.sync_copy(col_r.at[wid], colv)
    pltpu.sync_copy(ew_r.at[wid], ewv)

    def _edges(t, _):
        r = rowv[pl.ds(t * 16, 16)]
        c = colv[pl.ds(t * 16, 16)]
        w = ewv[pl.ds(t * 16, 16)]
        wm = jnp.where(r != c, w, jnp.zeros((16,), _f32))
        plsc.addupdate_scatter(degl, [c], wm)
        return 0
    lax.fori_loop(0, EPW // 16, _edges, 0)
    pltpu.sync_copy(degl, out.at[wid])


def _dis_body(dp, dis_o, dis2_o):
    deg = jnp.sum(dp[...], axis=0) + 1.0
    dis_o[...] = lax.rsqrt(deg)
    dis2_o[...] = 1.0 / deg


def _dis(degp):
    spec = pl.BlockSpec((NPAD // 128, 128), lambda: (0, 0))
    return pl.pallas_call(
        _dis_body,
        in_specs=[pl.BlockSpec((NW, NPAD // 128, 128), lambda: (0, 0, 0))],
        out_specs=[spec, spec],
        out_shape=[jax.ShapeDtypeStruct((NPAD // 128, 128), _f32),
                   jax.ShapeDtypeStruct((NPAD // 128, 128), _f32)],
    )(degp.reshape(NW, NPAD // 128, 128))


@functools.partial(
    pl.kernel,
    out_type=jax.ShapeDtypeStruct((NW, EPW), _f32),
    mesh=_mesh,
    compiler_params=_sc_params,
    scratch_types=[
        pltpu.VMEM((EPW,), _i32),
        pltpu.VMEM((EPW,), _i32),
        pltpu.VMEM((EPW,), _f32),
        pltpu.VMEM((EPW,), _f32),
        pltpu.VMEM((NPAD,), _f32),
    ],
)
def _norm_kernel(row_r, col_r, ew_r, dis_r, out, rowv, colv, ewv, normv, disv):
    cid = lax.axis_index("c")
    sid = lax.axis_index("s")
    wid = sid * 2 + cid
    pltpu.sync_copy(row_r.at[wid], rowv)
    pltpu.sync_copy(col_r.at[wid], colv)
    pltpu.sync_copy(ew_r.at[wid], ewv)
    pltpu.sync_copy(dis_r, disv)

    def _edges(t, _):
        r = rowv[pl.ds(t * 16, 16)]
        c = colv[pl.ds(t * 16, 16)]
        w = ewv[pl.ds(t * 16, 16)]
        a = plsc.load_gather(disv, [r])
        b = plsc.load_gather(disv, [c])
        wm = jnp.where(r != c, w, jnp.zeros((16,), _f32))
        normv[pl.ds(t * 16, 16)] = a * wm * b
        return 0
    lax.fori_loop(0, EPW // 16, _edges, 0)
    pltpu.sync_copy(normv, out.at[wid])


@functools.partial(
    pl.kernel,
    out_type=[jax.ShapeDtypeStruct((NPAD, D), _f32),
              jax.ShapeDtypeStruct((NPAD, D), _f32)],
    mesh=_mesh,
    compiler_params=_sc_params,
    scratch_types=[
        pltpu.VMEM((4, 3, K), _i32),
        pltpu.VMEM((K, D), _f32),
        pltpu.VMEM((K, D), _f32),
        pltpu.SemaphoreType.DMA,
        pltpu.SemaphoreType.DMA,
        pltpu.SemaphoreType.DMA,
        pltpu.SemaphoreType.DMA,
        pltpu.SemaphoreType.DMA,
        pltpu.SemaphoreType.DMA,
        pltpu.SemaphoreType.DMA,
        pltpu.SemaphoreType.DMA,
        pltpu.SemaphoreType.DMA,
        pltpu.VMEM_SHARED((NPAD, D), _f32),
    ],
)
def _hop_kernel(h_r, sh_r, ed_r, p0, p1, edb, rows0, rows1, esem0, esem1,
                esem2, esem3, gsem0, gsem1, ssem0, ssem1, seedsem, acc):
    cid = lax.axis_index("c")
    sid = lax.axis_index("s")
    wid = sid * 2 + cid
    npt = NPAD // 16
    rows = (rows0, rows1)
    esem = (esem0, esem1, esem2, esem3)
    gsem = (gsem0, gsem1)
    ssem = (ssem0, ssem1)

    pltpu.async_copy(sh_r.at[pl.ds(sid * npt, npt)],
                     acc.at[pl.ds(sid * npt, npt)], seedsem)

    def _start_ed(c, e):
        pltpu.async_copy(ed_r.at[wid, c], edb.at[e], esem[e])

    def _wait_ed(e):
        pltpu.make_async_copy(ed_r.at[wid, 0], edb.at[e], esem[e]).wait()

    def _start_g(e, s):
        pltpu.async_copy(h_r.at[edb.at[e, 0]], rows[s], gsem[s])

    def _wait_g(s):
        pltpu.make_async_copy(h_r.at[edb.at[0, 0]], rows[s], gsem[s]).wait()

    def _start_sc(s, e):
        pltpu.async_copy(rows[s], acc.at[edb.at[e, 1]], ssem[s], add=True)

    def _wait_sc(s):
        pltpu.make_async_copy(h_r.at[edb.at[0, 0]], rows[s], ssem[s]).wait()

    def _scale(s, e):
        buf = rows[s]

        def _grp(g, _):
            i0 = g * 4
            for u in range(4):
                nb = plsc.load_gather(
                    edb.at[e, 2], [jnp.full((16,), i0 + u, _i32)])
                nv = lax.bitcast_convert_type(nb, _f32)
                for j in range(D // 16):
                    sl = pl.ds(j * 16, 16)
                    buf[i0 + u, sl] = buf[i0 + u, sl] * nv
            return 0
        lax.fori_loop(0, K // 4, _grp, 0)

    def _body(c, s, e):
        _wait_ed((e + 1) % 4)
        _wait_sc(1 - s)
        _start_g((e + 1) % 4, 1 - s)
        _wait_g(s)
        _scale(s, e)
        _start_sc(s, e)
        _start_ed(c + 2, (e + 2) % 4)

    pltpu.sync_copy(ed_r.at[wid, 0], edb.at[0])
    _start_g(0, 0)
    _start_ed(1, 1)
    _wait_ed(1)
    _start_g(1, 1)
    _wait_g(0)
    _scale(0, 0)
    pltpu.make_async_copy(sh_r.at[pl.ds(sid * npt, npt)],
                          acc.at[pl.ds(sid * npt, npt)], seedsem).wait()
    plsc.subcore_barrier()
    _start_sc(0, 0)
    _start_ed(2, 2)
    _wait_ed(2)
    _wait_sc(0)
    _start_g(2, 0)
    _wait_g(1)
    _scale(1, 1)
    _start_sc(1, 1)
    _start_ed(3, 3)

    def _quad(g, _):
        c0 = 4 * g + 2
        _body(c0, 0, 2)
        _body(c0 + 1, 1, 3)
        _body(c0 + 2, 0, 0)
        _body(c0 + 3, 1, 1)
        return 0
    lax.fori_loop(0, (NCHUNK - 4) // 4, _quad, 0)
    _wait_ed(3)
    _wait_sc(1)
    _start_g(3, 1)
    _wait_g(0)
    _scale(0, 2)
    _start_sc(0, 2)
    _wait_g(1)
    _scale(1, 3)
    _start_sc(1, 3)
    _wait_sc(0)
    _wait_sc(1)
    plsc.subcore_barrier()

    @pl.when(cid == 0)
    def _():
        pltpu.sync_copy(acc.at[pl.ds(sid * npt, npt)],
                        p0.at[pl.ds(sid * npt, npt)])

    @pl.when(cid == 1)
    def _():
        pltpu.sync_copy(acc.at[pl.ds(sid * npt, npt)],
                        p1.at[pl.ds(sid * npt, npt)])


def _combine_body(p0, p1, d2, hout, shout):
    h = p0[...] + p1[...]
    hout[...] = h
    shout[...] = 0.5 * d2[...] * h


def _combine(p0, p1, d2b):
    BC = 1280
    spec = pl.BlockSpec((BC, D), lambda i: (i, 0))
    return pl.pallas_call(
        _combine_body,
        grid=(NPAD // BC,),
        in_specs=[spec, spec, spec],
        out_specs=[spec, spec],
        out_shape=[jax.ShapeDtypeStruct((NPAD, D), _f32),
                   jax.ShapeDtypeStruct((NPAD, D), _f32)],
    )(p0, p1, d2b)


def _channels_body(x_r, h1_r, h2_r, h4a_r, h4b_r, wch_r, bch_r, apl_r, acl_r,
                   apb_r, acb_r, wout_r, bout_r, out_r):
    act = jax.nn.gelu
    x = x_r[...]
    h1 = h1_r[...]
    h2 = h2_r[...]
    h4 = h4a_r[...] + h4b_r[...]
    pre_low = jnp.sum(act(x) * apl_r[...], axis=-1, keepdims=True)
    pre_band = jnp.sum(act(x) * apb_r[...], axis=-1, keepdims=True)
    zs = [x, h1, h2, h4, x - h1, h1 - h2, h2 - h4]
    outs = []
    scores = []
    for ci in range(7):
        o = (jnp.dot(zs[ci], wch_r[ci], preferred_element_type=_f32)
             + bch_r[ci:ci + 1, :])
        if ci < 4:
            s = jnp.sum(act(o) * acl_r[...], axis=-1, keepdims=True) + pre_low
        else:
            s = jnp.sum(act(o) * acb_r[...], axis=-1, keepdims=True) + pre_band
        scores.append(act(s))
        outs.append(o)
    S = jnp.concatenate(scores, axis=-1)
    m = jnp.max(S, axis=-1, keepdims=True)
    ex = jnp.exp(S - m)
    alpha = ex / jnp.sum(ex, axis=-1, keepdims=True)
    comb = outs[0] * alpha[:, 0:1]
    for ci in range(1, 7):
        comb = comb + outs[ci] * alpha[:, ci:ci + 1]
    out_r[...] = (jnp.dot(comb, wout_r[0:D], preferred_element_type=_f32)
                  + jnp.dot(x, wout_r[D:2 * D], preferred_element_type=_f32)
                  + bout_r[...])


def _channels(x, h1, h2, h4a, h4b, Wch, bch, apl, acl, apb, acb, Wout, bout):
    BN = 400
    row_spec = pl.BlockSpec((BN, D), lambda i: (i, 0))
    vec_spec = pl.BlockSpec((1, D), lambda i: (0, 0))
    return pl.pallas_call(
        _channels_body,
        grid=(N // BN,),
        in_specs=[
            row_spec, row_spec, row_spec, row_spec, row_spec,
            pl.BlockSpec((7, D, D), lambda i: (0, 0, 0)),
            pl.BlockSpec((7, D), lambda i: (0, 0)),
            vec_spec, vec_spec, vec_spec, vec_spec,
            pl.BlockSpec((2 * D, D), lambda i: (0, 0)),
            vec_spec,
        ],
        out_specs=row_spec,
        out_shape=jax.ShapeDtypeStruct((N, D), _f32),
    )(x, h1, h2, h4a, h4b, Wch, bch, apl, acl, apb, acb, Wout,
      bout.reshape(1, D))


def kernel(x, edge_index, edge_feats, W1, b1, W2, b2, Wch, bch, att_pre_low,
           att_channel_low, att_pre_band, att_channel_band, Wout, bout):
    row = edge_index[0]
    col = edge_index[1]
    row2d = row.reshape(NW, EPW)
    col2d = col.reshape(NW, EPW)

    ew = _edge_mlp(edge_feats, W1, b1, W2, b2)
    ew2d = ew.reshape(NW, EPW)

    degp = _deg_kernel(row2d, col2d, ew2d)
    dis_t, dis2_t = _dis(degp)
    dis = dis_t.reshape(NPAD)
    dis2 = dis2_t.reshape(NPAD)

    norm2d = _norm_kernel(row2d, col2d, ew2d, dis)
    npad_e = EPP - EPW
    pad = ((0, 0), (0, npad_e))
    pad_ids = jnp.broadcast_to(N + jnp.arange(npad_e, dtype=_i32)[None, :],
                               (NW, npad_e))
    rowp = jnp.concatenate([row2d, pad_ids], axis=1)
    colp = jnp.concatenate([col2d, pad_ids], axis=1)
    normp = jnp.pad(norm2d, pad)
    ed = jnp.concatenate(
        [rowp.reshape(NW, NCHUNK, 1, K),
         colp.reshape(NW, NCHUNK, 1, K),
         lax.bitcast_convert_type(normp, _i32).reshape(NW, NCHUNK, 1, K)],
        axis=2)

    x_pad = jnp.pad(x, ((0, NPAD - N), (0, 0)))
    d2b = jnp.broadcast_to(dis2[:, None], (NPAD, D))
    h = x_pad
    sh = 0.5 * dis2[:, None] * x_pad
    hs = []
    for _ in range(MAX_R - 1):
        p0, p1 = _hop_kernel(h, sh, ed)
        h, sh = _combine(p0, p1, d2b)
        hs.append(h)
    p0, p1 = _hop_kernel(h, sh, ed)

    return _channels(x, hs[0], hs[1], p0, p1, Wch, bch, att_pre_low,
                     att_channel_low, att_pre_band, att_channel_band,
                     Wout, bout)
